# baseline (speedup 1.0000x reference)
import functools

import jax
import jax.numpy as jnp
from jax import lax
from jax.experimental import pallas as pl
from jax.experimental.pallas import tpu as pltpu
from jax.experimental.pallas import tpu_sc as plsc

B = 16384
F = 512
D = 32
NC = 2
NS = 16
NW = NC * NS
BPW = B // NW

BLK = 1024
BLK2 = 4096


def _sc_gather_body(uidx, pidx, nidx, uemb, iemb, ue_out, pe_out, ne_out,
                    idx_v, rows_v, sem):
    wid = lax.axis_index("s") * NC + lax.axis_index("c")
    base = wid * BPW

    def do(idx_hbm, table, out_hbm):
        pltpu.sync_copy(idx_hbm.at[pl.ds(base, BPW)], idx_v)
        pltpu.async_copy(table.at[idx_v], rows_v, sem).wait()
        pltpu.sync_copy(rows_v, out_hbm.at[pl.ds(base, BPW)])

    do(uidx, uemb, ue_out)
    do(pidx, iemb, pe_out)
    do(nidx, iemb, ne_out)


def _sc_gather(uidx, pidx, nidx, uemb, iemb):
    mesh = plsc.VectorSubcoreMesh(core_axis_name="c", subcore_axis_name="s")
    out = jax.ShapeDtypeStruct((B, D), jnp.float32)
    fn = functools.partial(
        pl.kernel,
        mesh=mesh,
        out_type=(out, out, out),
        scratch_types=[
            pltpu.VMEM((BPW,), jnp.int32),
            pltpu.VMEM((BPW, D), jnp.float32),
            pltpu.SemaphoreType.DMA,
        ],
        compiler_params=pltpu.CompilerParams(use_tc_tiling_on_sc=False),
    )(_sc_gather_body)
    return fn(uidx, pidx, nidx, uemb, iemb)


def _tca_body(uf, pf, nf, umap, imap, w1b, hp_out, hn_out):
    un = (uf[...] - 2.5) * 0.4
    pn = (pf[...] - 2.5) * 0.4
    nn = (nf[...] - 2.5) * 0.4
    um = jnp.dot(un, umap[...], preferred_element_type=jnp.float32)
    pm = jnp.dot(pn, imap[...], preferred_element_type=jnp.float32)
    nm = jnp.dot(nn, imap[...], preferred_element_type=jnp.float32)

    c_last = (((1,), (1,)), ((), ()))
    hp_out[...] = lax.dot_general(um * pm, w1b[...], c_last,
                                  preferred_element_type=jnp.float32)
    hn_out[...] = lax.dot_general(um * nm, w1b[...], c_last,
                                  preferred_element_type=jnp.float32)


def _tca(uf, pf, nf, umap, imap, w1b):
    grid = B // BLK
    feat_spec = pl.BlockSpec((BLK, F), lambda i: (i, 0))

    def full(shape):
        return pl.BlockSpec(shape, lambda i: tuple(0 for _ in shape))

    out = jax.ShapeDtypeStruct((B, 10), jnp.float32)
    return pl.pallas_call(
        _tca_body,
        grid=(grid,),
        in_specs=[feat_spec, feat_spec, feat_spec,
                  full((F, D)), full((F, D)), full((10, D))],
        out_specs=(pl.BlockSpec((BLK, 10), lambda i: (i, 0)),
                   pl.BlockSpec((BLK, 10), lambda i: (i, 0))),
        out_shape=(out, out),
    )(uf, pf, nf, umap, imap, w1b)


def _tcb_body(ue, pe, ne, hfp, hfn, w1a, b1, w2, out):
    i = pl.program_id(0)
    uip = ue[...] * pe[...]
    uin = ue[...] * ne[...]

    c_last = (((1,), (1,)), ((), ()))
    hp = lax.dot_general(uip, w1a[...], c_last,
                         preferred_element_type=jnp.float32)
    hp = jnp.maximum(hp + hfp[...] + b1[...], 0.0)
    hn = lax.dot_general(uin, w1a[...], c_last,
                         preferred_element_type=jnp.float32)
    hn = jnp.maximum(hn + hfn[...] + b1[...], 0.0)

    d = lax.dot_general(hp - hn, w2[...], c_last,
                        preferred_element_type=jnp.float32)
    part = jnp.sum(jnp.minimum(d, 0.0) - jnp.log1p(jnp.exp(-jnp.abs(d))))

    @pl.when(i == 0)
    def _():
        out[0, 0] = 0.0

    out[0, 0] += part


def _tcb(ue, pe, ne, hfp, hfn, w1a, b1, w2):
    grid = B // BLK2
    emb_spec = pl.BlockSpec((BLK2, D), lambda i: (i, 0))
    h_spec = pl.BlockSpec((BLK2, 10), lambda i: (i, 0))

    def full(shape):
        return pl.BlockSpec(shape, lambda i: tuple(0 for _ in shape))

    return pl.pallas_call(
        _tcb_body,
        grid=(grid,),
        in_specs=[emb_spec, emb_spec, emb_spec, h_spec, h_spec,
                  full((10, D)), full((1, 10)), full((1, 10))],
        out_specs=pl.BlockSpec((1, 1), lambda i: (0, 0),
                               memory_space=pltpu.SMEM),
        out_shape=jax.ShapeDtypeStruct((1, 1), jnp.float32),
    )(ue, pe, ne, hfp, hfn, w1a, b1, w2)


def kernel(user_batch, user_feature_batch, pos_item_batch,
           pos_item_feature_batch, neg_item_batch, neg_item_feature_batch,
           user_emb, item_emb, user_map, item_map,
           fc1_w, fc1_b, fc2_w, fc2_b):
    uidx = user_batch.astype(jnp.int32)
    pidx = pos_item_batch.astype(jnp.int32)
    nidx = neg_item_batch.astype(jnp.int32)

    ue, pe, ne = _sc_gather(uidx, pidx, nidx, user_emb, item_emb)
    hfp, hfn = _tca(user_feature_batch, pos_item_feature_batch,
                    neg_item_feature_batch, user_map, item_map,
                    fc1_w[:, D:])
    total = _tcb(ue, pe, ne, hfp, hfn, fc1_w[:, :D],
                 fc1_b.reshape(1, 10), fc2_w)
    return -total[0, 0] / B
